# baseline (speedup 1.0000x reference)
import functools

import jax
import jax.numpy as jnp
from jax import lax
from jax.experimental import pallas as pl
from jax.experimental.pallas import tpu as pltpu
from jax.experimental.pallas import tpu_sc as plsc

_VOCAB = 1000000
_D = 64
_B = 4096
_H = 200
_TOT = _B * _H
_NW = 32
_PER_W = _TOT // _NW
_CHUNK = 128
_NCH = _PER_W // _CHUNK


def _sc_gather(table, idx3):
    mesh = plsc.VectorSubcoreMesh(core_axis_name="c", subcore_axis_name="s")

    @functools.partial(
        pl.kernel,
        mesh=mesh,
        compiler_params=pltpu.CompilerParams(use_tc_tiling_on_sc=False),
        out_type=jax.ShapeDtypeStruct((_TOT, _D), jnp.float32),
        scratch_types=[
            pltpu.VMEM((_NCH, _CHUNK), jnp.int32),
            pltpu.VMEM((_CHUNK, _D), jnp.float32),
            pltpu.SemaphoreType.DMA,
        ],
    )
    def k(table_hbm, idx_hbm, out_hbm, idx_v, rows_v, gsem):
        wid = lax.axis_index("s") * 2 + lax.axis_index("c")
        base = wid * _PER_W
        pltpu.sync_copy(idx_hbm.at[wid], idx_v)

        def body(j, carry):
            pltpu.async_copy(table_hbm.at[idx_v.at[j]], rows_v, gsem).wait()
            pltpu.sync_copy(rows_v, out_hbm.at[pl.ds(base + j * _CHUNK, _CHUNK)])
            return carry

        lax.fori_loop(0, _NCH, body, 0)

    return k(table, idx3)


def _tc_mask(word_idx):
    def mk(idx_ref, o_ref):
        o_ref[...] = idx_ref[...] > 0

    return pl.pallas_call(
        mk,
        out_shape=jax.ShapeDtypeStruct((_B, _H), jnp.bool_),
    )(word_idx)


def kernel(word_idx, emb_table):
    idx3 = word_idx.reshape(_NW, _NCH, _CHUNK)
    emb = _sc_gather(emb_table, idx3)
    mask = _tc_mask(word_idx)
    return emb.reshape(_B, _H, _D), mask


# speedup vs baseline: 1.1138x; 1.1138x over previous
import functools

import jax
import jax.numpy as jnp
from jax import lax
from jax.experimental import pallas as pl
from jax.experimental.pallas import tpu as pltpu
from jax.experimental.pallas import tpu_sc as plsc

_VOCAB = 1000000
_D = 64
_B = 4096
_H = 200
_TOT = _B * _H
_NW = 32
_PER_W = _TOT // _NW
_CHUNK = 128
_NCH = _PER_W // _CHUNK


_K = 4
_NG = _NCH // _K
_HROWS = _K * _CHUNK


def _sc_gather(table, idx3):
    mesh = plsc.VectorSubcoreMesh(core_axis_name="c", subcore_axis_name="s")

    @functools.partial(
        pl.kernel,
        mesh=mesh,
        compiler_params=pltpu.CompilerParams(use_tc_tiling_on_sc=False),
        out_type=jax.ShapeDtypeStruct((_TOT, _D), jnp.float32),
        scratch_types=[
            pltpu.VMEM((_NCH, _CHUNK), jnp.int32),
            pltpu.VMEM((2, _HROWS, _D), jnp.float32),
            pltpu.SemaphoreType.DMA,
            pltpu.SemaphoreType.DMA,
        ],
    )
    def k(table_hbm, idx_hbm, out_hbm, idx_v, halves, gsem, ssem):
        wid = lax.axis_index("s") * 2 + lax.axis_index("c")
        base = wid * _PER_W
        pltpu.sync_copy(idx_hbm.at[wid], idx_v)

        def fire_g(g, h):
            for b in range(_K):
                pltpu.async_copy(
                    table_hbm.at[idx_v.at[g * _K + b]],
                    halves.at[h].at[pl.ds(b * _CHUNK, _CHUNK)],
                    gsem,
                )

        def wait_g(h):
            pltpu.make_async_copy(
                table_hbm.at[pl.ds(0, _HROWS)], halves.at[h], gsem
            ).wait()

        def out_slice(g):
            return out_hbm.at[pl.ds(base + g * _HROWS, _HROWS)]

        def fire_s(g, h):
            pltpu.async_copy(halves.at[h], out_slice(g), ssem)

        def wait_s(g, h):
            pltpu.make_async_copy(halves.at[h], out_slice(g), ssem).wait()

        fire_g(0, 0)
        fire_g(1, 1)

        def body(g2, carry):
            g = 2 * g2
            wait_g(0)
            fire_s(g, 0)
            wait_s(g, 0)
            fire_g(g + 2, 0)
            wait_g(1)
            fire_s(g + 1, 1)
            wait_s(g + 1, 1)
            fire_g(g + 3, 1)
            return carry

        lax.fori_loop(0, _NG // 2 - 1, body, 0)

        g = _NG - 2
        wait_g(0)
        fire_s(g, 0)
        wait_g(1)
        fire_s(g + 1, 1)
        wait_s(g, 0)
        wait_s(g + 1, 1)

    return k(table, idx3)


def _tc_mask(word_idx):
    def mk(idx_ref, o_ref):
        o_ref[...] = idx_ref[...] > 0

    return pl.pallas_call(
        mk,
        out_shape=jax.ShapeDtypeStruct((_B, _H), jnp.bool_),
    )(word_idx)


def kernel(word_idx, emb_table):
    idx3 = word_idx.reshape(_NW, _NCH, _CHUNK)
    emb = _sc_gather(emb_table, idx3)
    mask = _tc_mask(word_idx)
    return emb.reshape(_B, _H, _D), mask
